# baseline (speedup 1.0000x reference)
import functools

import jax
import jax.numpy as jnp
from jax import lax
from jax.experimental import pallas as pl
from jax.experimental.pallas import tpu as pltpu
from jax.experimental.pallas import tpu_sc as plsc

B, L = 4096, 200
D0, D1, D2 = 64, 32, 32
DOUT = D0 + D1 + D2
V1, V2 = 1000, 200
N = B * L
NC, NS = 2, 16
NW = NC * NS
PER_W = N // NW
CH = 128
STEPS = PER_W // CH
TOT_CHUNKS = N // CH


def _sc_encode(idm, t0, t1, t2):
  mesh = plsc.VectorSubcoreMesh(core_axis_name="c", subcore_axis_name="s")

  @functools.partial(
      pl.kernel, mesh=mesh,
      out_type=jax.ShapeDtypeStruct((N, DOUT), jnp.float32),
      compiler_params=pltpu.CompilerParams(use_tc_tiling_on_sc=False),
      scratch_types=[
          pltpu.VMEM((4, CH), jnp.int32),
          pltpu.VMEM((4, CH), jnp.int32),
          pltpu.VMEM((CH, D0), jnp.float32),
          pltpu.VMEM((CH, D0), jnp.float32),
          pltpu.VMEM((CH, DOUT), jnp.float32),
          pltpu.VMEM((CH, DOUT), jnp.float32),
          pltpu.VMEM((V1, D1), jnp.float32),
          pltpu.VMEM((V2, D2), jnp.float32),
          pltpu.SemaphoreType.DMA,
          pltpu.SemaphoreType.DMA,
          pltpu.SemaphoreType.DMA,
          pltpu.SemaphoreType.DMA,
          pltpu.SemaphoreType.DMA,
          pltpu.SemaphoreType.DMA,
      ],
  )
  def k(idm_h, t0_h, t1_h, t2_h, out_h,
        idm0, idm1, e0_0, e0_1, o_0, o_1, t1_l, t2_l,
        gs0, gs1, is0, is1, os0, os1):
    wid = lax.axis_index("s") * NC + lax.axis_index("c")
    c_base = wid * STEPS

    idm_b = (idm0, idm1)
    e0_b = (e0_0, e0_1)
    o_b = (o_0, o_1)
    gs_b = (gs0, gs1)
    is_b = (is0, is1)
    os_b = (os0, os1)

    def issue_gather(slot):
      pltpu.async_copy(t0_h.at[idm_b[slot].at[0]], e0_b[slot], gs_b[slot])

    def wait_gather(slot):
      pltpu.make_async_copy(t0_h.at[idm_b[slot].at[0]], e0_b[slot], gs_b[slot]).wait()

    def compute(slot):
      idm_p, e0_p, o_p = idm_b[slot], e0_b[slot], o_b[slot]

      def half_body(h, hcarry):
        base = h * 64
        for g in range(4):
          gb = base + g * 16
          mvi = idm_p[3, pl.ds(gb, 16)]
          mv = lax.bitcast_convert_type(mvi, jnp.float32)
          iv1 = idm_p[1, pl.ds(gb, 16)]
          iv2 = idm_p[2, pl.ds(gb, 16)]
          for j in range(16):
            i = gb + j
            m = jnp.broadcast_to(mv[j], (16,))
            id1 = iv1[j]
            id2 = iv2[j]
            for c in range(D0 // 16):
              o_p[i, pl.ds(c * 16, 16)] = e0_p[i, pl.ds(c * 16, 16)] * m
            for c in range(D1 // 16):
              o_p[i, pl.ds(D0 + c * 16, 16)] = t1_l[id1, pl.ds(c * 16, 16)] * m
            for c in range(D2 // 16):
              o_p[i, pl.ds(D0 + D1 + c * 16, 16)] = t2_l[id2, pl.ds(c * 16, 16)] * m
        return hcarry

      lax.fori_loop(0, CH // 64, half_body, 0)

    pltpu.sync_copy(t1_h, t1_l)
    pltpu.sync_copy(t2_h, t2_l)

    pltpu.sync_copy(idm_h.at[c_base], idm0)
    issue_gather(0)
    pltpu.async_copy(idm_h.at[c_base + 1], idm1, is1)

    def stage(s, p, q):
      @pl.when(s + 1 < STEPS)
      def _():
        pltpu.make_async_copy(idm_h.at[c_base], idm_b[q], is_b[q]).wait()
        issue_gather(q)

      wait_gather(p)

      @pl.when(s >= 2)
      def _():
        pltpu.make_async_copy(o_b[p], out_h.at[pl.ds(0, CH)], os_b[p]).wait()

      compute(p)

      @pl.when(s + 2 < STEPS)
      def _():
        pltpu.async_copy(idm_h.at[c_base + s + 2], idm_b[p], is_b[p])

      pltpu.async_copy(o_b[p], out_h.at[pl.ds((c_base + s) * CH, CH)], os_b[p])

    def pair_body(s2, carry):
      stage(2 * s2, 0, 1)
      stage(2 * s2 + 1, 1, 0)
      return carry

    lax.fori_loop(0, STEPS // 2, pair_body, 0)

    pltpu.make_async_copy(o_0, out_h.at[pl.ds(0, CH)], os0).wait()
    pltpu.make_async_copy(o_1, out_h.at[pl.ds(0, CH)], os1).wait()

  return k(idm, t0, t1, t2)


def kernel(positional_ids_0, positional_ids_1, positional_ids_2,
           attention_mask, table0, table1, table2):
  mbits = lax.bitcast_convert_type(attention_mask, jnp.int32)
  idm = jnp.stack([
      positional_ids_0.reshape(TOT_CHUNKS, CH),
      positional_ids_1.reshape(TOT_CHUNKS, CH),
      positional_ids_2.reshape(TOT_CHUNKS, CH),
      mbits.reshape(TOT_CHUNKS, CH),
  ], axis=1)
  out = _sc_encode(idm, table0, table1, table2)
  return out.reshape(B, L, DOUT)


# speedup vs baseline: 2.7943x; 2.7943x over previous
import functools

import jax
import jax.numpy as jnp
from jax import lax
from jax.experimental import pallas as pl
from jax.experimental.pallas import tpu as pltpu
from jax.experimental.pallas import tpu_sc as plsc

B, L = 4096, 200
D0, D1, D2 = 64, 32, 32
DOUT = D0 + D1 + D2
N = B * L
NC, NS = 2, 16
NW = NC * NS
PER_W = N // NW
CH = 128
STEPS = PER_W // CH
TOT_CHUNKS = N // CH


def _sc_encode(idm, t0, t1, t2):
  mesh = plsc.VectorSubcoreMesh(core_axis_name="c", subcore_axis_name="s")

  @functools.partial(
      pl.kernel, mesh=mesh,
      out_type=jax.ShapeDtypeStruct((N, DOUT), jnp.float32),
      compiler_params=pltpu.CompilerParams(use_tc_tiling_on_sc=False),
      scratch_types=[
          pltpu.VMEM((4, CH), jnp.int32),
          pltpu.VMEM((4, CH), jnp.int32),
          pltpu.VMEM((CH, D0), jnp.float32),
          pltpu.VMEM((CH, D0), jnp.float32),
          pltpu.VMEM((CH, D1), jnp.float32),
          pltpu.VMEM((CH, D1), jnp.float32),
          pltpu.VMEM((CH, D2), jnp.float32),
          pltpu.VMEM((CH, D2), jnp.float32),
          pltpu.VMEM((CH, DOUT), jnp.float32),
          pltpu.VMEM((CH, DOUT), jnp.float32),
          pltpu.SemaphoreType.DMA,
          pltpu.SemaphoreType.DMA,
          pltpu.SemaphoreType.DMA,
          pltpu.SemaphoreType.DMA,
          pltpu.SemaphoreType.DMA,
          pltpu.SemaphoreType.DMA,
      ],
  )
  def k(idm_h, t0_h, t1_h, t2_h, out_h,
        idm0, idm1, e0_0, e0_1, e1_0, e1_1, e2_0, e2_1, o_0, o_1,
        gs0, gs1, is0, is1, os0, os1):
    wid = lax.axis_index("s") * NC + lax.axis_index("c")
    c_base = wid * STEPS

    idm_b = (idm0, idm1)
    e0_b = (e0_0, e0_1)
    e1_b = (e1_0, e1_1)
    e2_b = (e2_0, e2_1)
    o_b = (o_0, o_1)
    gs_b = (gs0, gs1)
    is_b = (is0, is1)
    os_b = (os0, os1)

    def issue_gathers(slot):
      pltpu.async_copy(t0_h.at[idm_b[slot].at[0]], e0_b[slot], gs_b[slot])
      pltpu.async_copy(t1_h.at[idm_b[slot].at[1]], e1_b[slot], gs_b[slot])
      pltpu.async_copy(t2_h.at[idm_b[slot].at[2]], e2_b[slot], gs_b[slot])

    def wait_gathers(slot):
      pltpu.make_async_copy(t0_h.at[idm_b[slot].at[0]], e0_b[slot], gs_b[slot]).wait()
      pltpu.make_async_copy(t1_h.at[idm_b[slot].at[1]], e1_b[slot], gs_b[slot]).wait()
      pltpu.make_async_copy(t2_h.at[idm_b[slot].at[2]], e2_b[slot], gs_b[slot]).wait()

    def compute(slot):
      idm_p, e0_p, e1_p, e2_p, o_p = (
          idm_b[slot], e0_b[slot], e1_b[slot], e2_b[slot], o_b[slot])
      for g in range(CH // 16):
        mvi = idm_p[3, pl.ds(g * 16, 16)]
        mv = lax.bitcast_convert_type(mvi, jnp.float32)
        for j in range(16):
          i = g * 16 + j
          m = jnp.broadcast_to(mv[j], (16,))
          for c in range(D0 // 16):
            o_p[i, pl.ds(c * 16, 16)] = e0_p[i, pl.ds(c * 16, 16)] * m
          for c in range(D1 // 16):
            o_p[i, pl.ds(D0 + c * 16, 16)] = e1_p[i, pl.ds(c * 16, 16)] * m
          for c in range(D2 // 16):
            o_p[i, pl.ds(D0 + D1 + c * 16, 16)] = e2_p[i, pl.ds(c * 16, 16)] * m

    pltpu.sync_copy(idm_h.at[c_base], idm0)
    issue_gathers(0)
    pltpu.async_copy(idm_h.at[c_base + 1], idm1, is1)

    def stage(s, p, q):
      @pl.when(s + 1 < STEPS)
      def _():
        pltpu.make_async_copy(idm_h.at[c_base], idm_b[q], is_b[q]).wait()
        issue_gathers(q)

      wait_gathers(p)

      @pl.when(s >= 2)
      def _():
        pltpu.make_async_copy(o_b[p], out_h.at[pl.ds(0, CH)], os_b[p]).wait()

      compute(p)

      @pl.when(s + 2 < STEPS)
      def _():
        pltpu.async_copy(idm_h.at[c_base + s + 2], idm_b[p], is_b[p])

      pltpu.async_copy(o_b[p], out_h.at[pl.ds((c_base + s) * CH, CH)], os_b[p])

    def pair_body(s2, carry):
      stage(2 * s2, 0, 1)
      stage(2 * s2 + 1, 1, 0)
      return carry

    lax.fori_loop(0, STEPS // 2, pair_body, 0)

    pltpu.make_async_copy(o_0, out_h.at[pl.ds(0, CH)], os0).wait()
    pltpu.make_async_copy(o_1, out_h.at[pl.ds(0, CH)], os1).wait()

  return k(idm, t0, t1, t2)


def kernel(positional_ids_0, positional_ids_1, positional_ids_2,
           attention_mask, table0, table1, table2):
  V1, V2 = table1.shape[0], table2.shape[0]
  t1_rep = jnp.tile(table1, (NW, 1))
  t2_rep = jnp.tile(table2, (NW, 1))
  toff = jnp.arange(NW, dtype=jnp.int32)[:, None]
  ids1_adj = (positional_ids_1.reshape(NW, PER_W) + toff * V1).reshape(TOT_CHUNKS, CH)
  ids2_adj = (positional_ids_2.reshape(NW, PER_W) + toff * V2).reshape(TOT_CHUNKS, CH)
  mbits = lax.bitcast_convert_type(attention_mask, jnp.int32)
  idm = jnp.stack([
      positional_ids_0.reshape(TOT_CHUNKS, CH),
      ids1_adj,
      ids2_adj,
      mbits.reshape(TOT_CHUNKS, CH),
  ], axis=1)
  out = _sc_encode(idm, table0, t1_rep, t2_rep)
  return out.reshape(B, L, DOUT)
